# baseline (speedup 1.0000x reference)
import functools

import jax
import jax.numpy as jnp
from jax import lax
from jax.experimental import pallas as pl
from jax.experimental.pallas import tpu as pltpu
from jax.experimental.pallas import tpu_sc as plsc


_NC = 2
_NS = 16
_NW = _NC * _NS

_LANES = 16
_CW = 256
_POS_BITS = 14
_SENTINEL = 0x7FFFFFFF


def _make_kernel(n_nodes: int, batch: int, dim: int):
  assert dim == 64 and batch % _LANES == 0 and batch <= (1 << _POS_BITS)
  nch = -(-n_nodes // _CW)
  n_full = (nch - 1) * _CW
  tail_w = n_nodes - n_full
  tail_owner = (nch - 1) % _NW
  tail_g = (nch - 1) // _NW
  g_max = -(-nch // _NW)
  slots = g_max * _CW
  assert (slots << _POS_BITS) < 2**31
  list_len = batch + _LANES

  mesh = plsc.VectorSubcoreMesh(core_axis_name="c", subcore_axis_name="s",
                                num_cores=_NC, num_subcores=_NS)

  @functools.partial(
      pl.kernel,
      mesh=mesh,
      out_type=(jax.ShapeDtypeStruct((dim, n_nodes), jnp.float32),
                jax.ShapeDtypeStruct((dim * tail_w,), jnp.float32)),
      compiler_params=pltpu.CompilerParams(needs_layout_passes=False),
      scratch_types=[
          pltpu.VMEM((batch,), jnp.int32),
          pltpu.VMEM((list_len,), jnp.int32),
          pltpu.VMEM((list_len,), jnp.int32),
          pltpu.VMEM((slots,), jnp.int32),
          pltpu.VMEM((2 * _LANES,), jnp.int32),
          pltpu.VMEM((_CW,), jnp.int32),
          pltpu.VMEM((_CW,), jnp.int32),
          pltpu.VMEM((dim * _LANES,), jnp.int32),
          pltpu.VMEM((dim * _LANES,), jnp.float32),
          pltpu.VMEM((dim, _CW), jnp.float32),
          pltpu.VMEM((dim * tail_w,), jnp.float32),
          pltpu.SemaphoreType.DMA,
          pltpu.SemaphoreType.DMA,
      ],
  )
  def body(mem_hbm, idx_hbm, valf_hbm, tailf_hbm, out_hbm, tout_hbm,
           idx_v, pos_list, slot_list, board, tmp, cpos, ccol, ib, stg, blk,
           tailv, sem, sem2):
    wid = lax.axis_index("s") * _NC + lax.axis_index("c")
    lane = lax.iota(jnp.int32, _LANES)

    pltpu.sync_copy(idx_hbm, idx_v)

    def _zero(j, _):
      board[pl.ds(j * _LANES, _LANES)] = jnp.full((_LANES,), -1, jnp.int32)
      return 0
    lax.fori_loop(0, slots // _LANES, _zero, 0)
    tmp[pl.ds(_LANES, _LANES)] = jnp.full((_LANES,), -1, jnp.int32)

    def _compact(i, cnt):
      v = idx_v[pl.ds(i * _LANES, _LANES)]
      m = ((v >> 8) & (_NW - 1)) == wid
      slot = ((v >> 13) << 8) | (v & (_CW - 1))
      plsc.store_compressed(pos_list.at[pl.ds(cnt, _LANES)],
                            lane + i * _LANES, mask=m)
      plsc.store_compressed(slot_list.at[pl.ds(cnt, _LANES)], slot, mask=m)
      return cnt + jnp.sum(m.astype(jnp.int32))
    k = lax.fori_loop(0, batch // _LANES, _compact, jnp.int32(0))

    def _claim(i, _):
      off = i * _LANES
      pos_v = pos_list[pl.ds(off, _LANES)]
      sl = slot_list[pl.ds(off, _LANES)]
      valid = (lane + off) < k
      comp = jnp.where(valid, (sl << _POS_BITS) | pos_v, jnp.int32(_SENTINEL))
      s = jnp.sort(comp)
      invalid_s = s == jnp.int32(_SENTINEL)
      sls = jnp.where(invalid_s, -2, s >> _POS_BITS)
      poss = s & ((1 << _POS_BITS) - 1)
      tmp[pl.ds(0, _LANES)] = sls
      nxt = tmp[pl.ds(1, _LANES)]
      is_last = (sls != nxt) & ~invalid_s
      safe = jnp.where(is_last, sls, 0)
      cur = plsc.load_gather(board, [safe])
      plsc.store_scatter(board, [safe], poss, mask=is_last & (poss > cur))
      return 0
    lax.fori_loop(0, (k + _LANES - 1) // _LANES, _claim, 0)

    def _collect(g):
      cnt = jnp.int32(0)
      for j in range(_CW // _LANES):
        b = board[pl.ds(g * _CW + j * _LANES, _LANES)]
        m = b >= 0
        plsc.store_compressed(cpos.at[pl.ds(cnt, _LANES)], b, mask=m)
        plsc.store_compressed(ccol.at[pl.ds(cnt, _LANES)],
                              lane + j * _LANES, mask=m)
        cnt = cnt + jnp.sum(m.astype(jnp.int32))
      return cnt

    def _patch_into(cnt, apply):
      def _patch(j2, _):
        valid = (lane + j2 * _LANES) < cnt
        p = jnp.where(valid, cpos[pl.ds(j2 * _LANES, _LANES)], 0)
        c = jnp.where(valid, ccol[pl.ds(j2 * _LANES, _LANES)], 0)
        for dd in range(dim):
          ib[pl.ds(dd * _LANES, _LANES)] = p * dim + dd
        pltpu.async_copy(valf_hbm.at[ib], stg, sem2).wait()
        for dd in range(dim):
          v = stg[pl.ds(dd * _LANES, _LANES)]
          apply(dd, c, v, valid)
        return 0
      lax.fori_loop(0, (cnt + _LANES - 1) // _LANES, _patch, 0)

    def _chunk(g, _):
      c0 = (wid + _NW * g) * _CW
      pltpu.sync_copy(mem_hbm.at[:, pl.ds(c0, _CW)], blk)
      cnt = _collect(g)

      def _apply_blk(dd, c, v, valid):
        plsc.store_scatter(blk, [jnp.full((_LANES,), dd, jnp.int32), c],
                           v, mask=valid)
      _patch_into(cnt, _apply_blk)
      pltpu.sync_copy(blk, out_hbm.at[:, pl.ds(c0, _CW)])
      return 0
    n_my_full = (nch - 2 - wid) // _NW + 1
    lax.fori_loop(0, n_my_full, _chunk, 0)

    @pl.when(wid == tail_owner)
    def _tail():
      pltpu.sync_copy(tailf_hbm, tailv)
      cnt = _collect(tail_g)

      def _apply_tail(dd, c, v, valid):
        plsc.store_scatter(tailv, [c * dim + dd], v, mask=valid)
      _patch_into(cnt, _apply_tail)
      pltpu.sync_copy(tailv, tout_hbm)

  return body


def kernel(memory, node_idxs, values):
  n_nodes, dim = memory.shape
  batch, = node_idxs.shape
  nch = -(-n_nodes // _CW)
  n_full = (nch - 1) * _CW
  out_t, tail_flat = _make_kernel(n_nodes, batch, dim)(
      memory.T, node_idxs, values.reshape(-1),
      memory[n_full:].reshape(-1))
  tail_t = tail_flat.reshape(n_nodes - n_full, dim).T
  return lax.dynamic_update_slice(out_t, tail_t, (0, n_full)).T
